# baseline (speedup 1.0000x reference)
import jax
import jax.numpy as jnp
from jax.experimental import pallas as pl
from jax.experimental.pallas import tpu as pltpu
from jax.experimental.pallas import tpu_sc as plsc

VOCAB = 50
POS_DIM = 64
SEQ = 200


def _sc_gather(rep_table, indices):
    batch, seq = indices.shape
    mesh = plsc.VectorSubcoreMesh(core_axis_name="core", subcore_axis_name="subcore")

    @pl.kernel(
        out_type=jax.ShapeDtypeStruct((batch, seq, 2 * POS_DIM), rep_table.dtype),
        mesh=mesh,
    )
    def gather_kernel(table_hbm, idx_hbm, out_hbm):
        def body(idx_vmem, out_vmem):
            idx_row = idx_vmem.at[0]
            dst = out_vmem.at[0]
            pltpu.sync_copy(
                table_hbm.at[idx_row.at[pl.ds(0, 128)]],
                dst.at[pl.ds(0, 128)],
            )
            pltpu.sync_copy(
                table_hbm.at[idx_row.at[pl.ds(128, seq - 128)]],
                dst.at[pl.ds(128, seq - 128)],
            )

        pltpu.emit_pipeline(
            body,
            grid=(batch,),
            in_specs=[pl.BlockSpec((1, seq), index_map=lambda i: (i, 0))],
            out_specs=[
                pl.BlockSpec((1, seq, 2 * POS_DIM), index_map=lambda i: (i, 0, 0))
            ],
            core_axis_name=("core", "subcore"),
            dimension_semantics=(pltpu.PARALLEL,),
        )(idx_hbm, out_hbm)

    return gather_kernel(rep_table, indices)


def kernel(indices, table):
    rep_table = jnp.concatenate([table, table], axis=1)
    wide = _sc_gather(rep_table, indices.astype(jnp.int32))
    return wide[:, :, :POS_DIM]


# speedup vs baseline: 1.8751x; 1.8751x over previous
import jax
import jax.numpy as jnp
from jax.experimental import pallas as pl
from jax.experimental.pallas import tpu as pltpu
from jax.experimental.pallas import tpu_sc as plsc

VOCAB = 50
POS_DIM = 64
WINDOW = 128


def _sc_gather(pair_table, idx_flat, n_pairs):
    mesh = plsc.VectorSubcoreMesh(core_axis_name="core", subcore_axis_name="subcore")

    @pl.kernel(
        out_type=jax.ShapeDtypeStruct((n_pairs, 2 * POS_DIM), pair_table.dtype),
        mesh=mesh,
    )
    def gather_kernel(table_hbm, idx_hbm, out_hbm):
        def body(idx_vmem, out_vmem):
            pltpu.sync_copy(table_hbm.at[idx_vmem.at[0]], out_vmem)

        pltpu.emit_pipeline(
            body,
            grid=(n_pairs // WINDOW,),
            in_specs=[pl.BlockSpec((1, WINDOW), index_map=lambda i: (0, i))],
            out_specs=[
                pl.BlockSpec((WINDOW, 2 * POS_DIM), index_map=lambda i: (i, 0))
            ],
            core_axis_name=("core", "subcore"),
            dimension_semantics=(pltpu.PARALLEL,),
        )(idx_hbm, out_hbm)

    return gather_kernel(pair_table, idx_flat)


def kernel(indices, table):
    batch, seq_len = indices.shape
    half = batch // 2
    n_pairs = half * seq_len

    pair_table = jnp.concatenate(
        [
            jnp.broadcast_to(table[:, None, :], (VOCAB, VOCAB, POS_DIM)),
            jnp.broadcast_to(table[None, :, :], (VOCAB, VOCAB, POS_DIM)),
        ],
        axis=-1,
    ).reshape(VOCAB * VOCAB, 2 * POS_DIM)

    idx = indices.astype(jnp.int32)
    pair_idx = (idx[:half] * VOCAB + idx[half:]).reshape(1, n_pairs)

    wide = _sc_gather(pair_table, pair_idx, n_pairs)
    wide = wide.reshape(half, seq_len, 2 * POS_DIM)
    return jnp.concatenate(
        [wide[:, :, :POS_DIM], wide[:, :, POS_DIM:]], axis=0
    )
